# baseline (speedup 1.0000x reference)
import functools

import jax
import jax.numpy as jnp
from jax import lax
from jax.experimental import pallas as pl
from jax.experimental.pallas import tpu as pltpu
from jax.experimental.pallas import tpu_sc as plsc

N_ROWS = 163842
D = 256
B = (N_ROWS + 6) // 4
NC, NS = 2, 16
NW = NC * NS
RG = 16
SL = 7 * RG
NSTR = 80
RPW = NSTR * RG
B_MAIN = NW * RPW
SCALE = 1.0 / 7.0
L = 16
NBUF = 4
HG = 8
NOB = 2


def _pack_indices(neigh_orders):
    idx = neigh_orders[: B_MAIN * 7].reshape(NW, NSTR * SL).astype(jnp.int32)
    tail = neigh_orders[B_MAIN * 7 : B * 7]
    tail = jnp.concatenate([tail, tail[:2]]).astype(jnp.int32)
    return idx, tail


@functools.partial(
    pl.kernel,
    out_type=jax.ShapeDtypeStruct((B, D), jnp.float32),
    mesh=plsc.VectorSubcoreMesh(core_axis_name="c", subcore_axis_name="s"),
    scratch_types=[
        pltpu.VMEM((NSTR * SL,), jnp.int32),
        pltpu.VMEM((16,), jnp.int32),
        pltpu.VMEM((NBUF * SL, D), jnp.float32),
        pltpu.VMEM((NOB * HG, D), jnp.float32),
        pltpu.SemaphoreType.DMA,
        pltpu.SemaphoreType.DMA,
    ],
)
def _pool_kernel(x_hbm, idx_hbm, tail_hbm, out_hbm, idx_v, tail_v, gbuf, obuf,
                 sem, osem):
    wid = lax.axis_index("s") * NC + lax.axis_index("c")
    pltpu.sync_copy(idx_hbm.at[wid], idx_v)
    base = wid * RPW

    def fire(s, slot):
        return pltpu.async_copy(
            x_hbm.at[idx_v.at[pl.ds(s * SL, SL)]], gbuf.at[pl.ds(slot * SL, SL)], sem
        )

    def wait(s, slot):
        pltpu.make_async_copy(
            x_hbm.at[idx_v.at[pl.ds(s * SL, SL)]], gbuf.at[pl.ds(slot * SL, SL)], sem
        ).wait()

    def owait():
        pltpu.make_async_copy(
            obuf.at[pl.ds(0, HG)], out_hbm.at[pl.ds(base, HG)], osem
        ).wait()

    for i in range(NBUF - 1):
        fire(i, i)

    def step(s, carry):
        slot = lax.rem(s, NBUF)
        wait(s, slot)

        @pl.when(s < NSTR - (NBUF - 1))
        def _prefetch():
            fire(s + NBUF - 1, lax.rem(s + NBUF - 1, NBUF))

        @pl.when(s >= 1)
        def _drain_stores():
            owait()
            owait()

        g0 = slot * SL
        ob0 = lax.rem(2 * s, NOB) * HG
        ob1 = lax.rem(2 * s + 1, NOB) * HG

        def half(h, ob):
            @plsc.parallel_loop(0, HG, unroll=2)
            def _reduce(r):
                rb = g0 + 7 * (h * HG + r)
                for l in range(D // L):
                    sl = pl.ds(l * L, L)
                    acc = gbuf[rb, sl]
                    for j in range(1, 7):
                        acc = acc + gbuf[rb + j, sl]
                    obuf[ob + r, sl] = acc * jnp.float32(SCALE)

            pltpu.async_copy(
                obuf.at[pl.ds(ob, HG)],
                out_hbm.at[pl.ds(base + s * RG + h * HG, HG)],
                osem,
            )

        half(0, ob0)
        half(1, ob1)
        return carry

    lax.fori_loop(0, NSTR, step, 0, unroll=False)
    owait()
    owait()

    @pl.when(wid == 0)
    def _tail():
        pltpu.sync_copy(tail_hbm, tail_v)
        pltpu.async_copy(x_hbm.at[tail_v], gbuf.at[pl.ds(0, 16)], sem).wait()
        for r in range(2):
            for l in range(D // L):
                sl = pl.ds(l * L, L)
                acc = gbuf[7 * r, sl]
                for j in range(1, 7):
                    acc = acc + gbuf[7 * r + j, sl]
                obuf[r, sl] = acc * jnp.float32(SCALE)
        pltpu.sync_copy(obuf.at[pl.ds(0, 2)], out_hbm.at[pl.ds(B_MAIN, 2)])


def kernel(x, neigh_orders):
    idx, tail = _pack_indices(neigh_orders)
    return _pool_kernel(x, idx, tail)


# speedup vs baseline: 2.1409x; 2.1409x over previous
import functools

import jax
import jax.numpy as jnp
from jax import lax
from jax.experimental import pallas as pl
from jax.experimental.pallas import tpu as pltpu
from jax.experimental.pallas import tpu_sc as plsc

N_ROWS = 163842
D = 256
B = (N_ROWS + 6) // 4
NC, NS = 2, 16
NW = NC * NS
RG = 16
SL = 7 * RG
NSTR = 80
RPW = NSTR * RG
B_MAIN = NW * RPW
SCALE = 1.0 / 7.0
L = 16
NBUF = 3


def _pack_indices(neigh_orders):
    idx = neigh_orders[: B_MAIN * 7].reshape(NW, NSTR, SL).astype(jnp.int32)
    tail = neigh_orders[B_MAIN * 7 : B * 7]
    tail = jnp.concatenate([tail, tail[:2]]).astype(jnp.int32)
    return idx, tail


@functools.partial(
    pl.kernel,
    out_type=jax.ShapeDtypeStruct((B, D), jnp.float32),
    mesh=plsc.VectorSubcoreMesh(core_axis_name="c", subcore_axis_name="s"),
    scratch_types=[
        pltpu.VMEM((NSTR, SL), jnp.int32),
        pltpu.VMEM((16,), jnp.int32),
        pltpu.VMEM((NBUF * SL, D), jnp.float32),
        pltpu.VMEM((2 * RG, D), jnp.float32),
        pltpu.SemaphoreType.DMA,
        pltpu.SemaphoreType.DMA,
    ],
)
def _pool_kernel(x_hbm, idx_hbm, tail_hbm, out_hbm, idx_v, tail_v, gbuf, obuf,
                 sem, osem):
    wid = lax.axis_index("s") * NC + lax.axis_index("c")
    pltpu.sync_copy(idx_hbm.at[wid], idx_v)
    base = wid * RPW

    def fire(s, slot):
        return pltpu.async_copy(
            x_hbm.at[idx_v.at[s]], gbuf.at[pl.ds(slot * SL, SL)], sem
        )

    def wait(s, slot):
        pltpu.make_async_copy(
            x_hbm.at[idx_v.at[s]], gbuf.at[pl.ds(slot * SL, SL)], sem
        ).wait()

    def owait():
        pltpu.make_async_copy(
            obuf.at[pl.ds(0, RG)], out_hbm.at[pl.ds(base, RG)], osem
        ).wait()

    for i in range(NBUF - 1):
        fire(i, i)

    def step(s, carry):
        slot = lax.rem(s, NBUF)
        wait(s, slot)

        @pl.when(s < NSTR - (NBUF - 1))
        def _prefetch():
            fire(s + NBUF - 1, lax.rem(s + NBUF - 1, NBUF))

        @pl.when(s >= 2)
        def _drain_store():
            owait()

        g0 = slot * SL
        o0 = lax.rem(s, 2) * RG

        @plsc.parallel_loop(0, RG, unroll=2)
        def _reduce(r):
            rb = g0 + 7 * r
            for l in range(D // L):
                sl = pl.ds(l * L, L)
                acc = gbuf[rb, sl]
                for j in range(1, 7):
                    acc = acc + gbuf[rb + j, sl]
                obuf[o0 + r, sl] = acc * jnp.float32(SCALE)

        pltpu.async_copy(
            obuf.at[pl.ds(o0, RG)],
            out_hbm.at[pl.ds(base + s * RG, RG)],
            osem,
        )
        return carry

    lax.fori_loop(0, NSTR, step, 0, unroll=False)
    owait()
    owait()

    @pl.when(wid == 0)
    def _tail():
        pltpu.sync_copy(tail_hbm, tail_v)
        pltpu.async_copy(x_hbm.at[tail_v], gbuf.at[pl.ds(0, 16)], sem).wait()
        for r in range(2):
            for l in range(D // L):
                sl = pl.ds(l * L, L)
                acc = gbuf[7 * r, sl]
                for j in range(1, 7):
                    acc = acc + gbuf[7 * r + j, sl]
                obuf[r, sl] = acc * jnp.float32(SCALE)
        pltpu.sync_copy(obuf.at[pl.ds(0, 2)], out_hbm.at[pl.ds(B_MAIN, 2)])


def kernel(x, neigh_orders):
    idx, tail = _pack_indices(neigh_orders)
    return _pool_kernel(x, idx, tail)
